# baseline (speedup 1.0000x reference)
import functools

import jax
import jax.numpy as jnp
from jax import lax
from jax.experimental import pallas as pl
from jax.experimental.pallas import tpu as pltpu
from jax.experimental.pallas import tpu_sc as plsc

NC = 2
NS = 16
L = 16
NW = NC * NS

B = 16384
D = 64
BPW = B // NW
IDX_CHUNK = 128
N_IDX = BPW // IDX_CHUNK
GROUPS_PER_STEP = 4
ROWS_PER_STEP = GROUPS_PER_STEP * L
N_STEPS = BPW // ROWS_PER_STEP


def _sc_body(users_hbm, movies_hbm, ratings_hbm, utab_hbm, mtab_hbm,
             params_hbm, out_hbm, part_hbm,
             idx_u, idx_m, rows_u, rows_m, rat_v, out_v, params_v, part_v,
             sem):
    wid = lax.axis_index("s") * NC + lax.axis_index("c")

    pltpu.sync_copy(params_hbm, params_v)
    pltpu.sync_copy(users_hbm.at[wid], idx_u)
    pltpu.sync_copy(movies_hbm.at[wid], idx_m)
    pltpu.sync_copy(ratings_hbm.at[wid], rat_v)

    for j in range(N_IDX):
        pltpu.async_copy(utab_hbm.at[idx_u.at[j]],
                         rows_u.at[pl.ds(j * IDX_CHUNK, IDX_CHUNK)], sem).wait()
    for j in range(N_IDX):
        pltpu.async_copy(mtab_hbm.at[idx_m.at[j]],
                         rows_m.at[pl.ds(j * IDX_CHUNK, IDX_CHUNK)], sem).wait()

    iota = lax.iota(jnp.int32, L)
    zero = jnp.zeros((L,), jnp.float32)

    def pbc(d):
        return params_v[pl.ds(d * L, L)]

    bias = pbc(2 * D)

    def step(c, lacc):
        base = c * ROWS_PER_STEP
        accs = [bias for _ in range(GROUPS_PER_STEP)]
        ids = [base + q * L + iota for q in range(GROUPS_PER_STEP)]
        for d in range(D):
            wu = pbc(d)
            dcol = jnp.full((L,), d, jnp.int32)
            for q in range(GROUPS_PER_STEP):
                accs[q] = accs[q] + plsc.load_gather(rows_u, [ids[q], dcol]) * wu
        for d in range(D):
            wm = pbc(D + d)
            dcol = jnp.full((L,), d, jnp.int32)
            for q in range(GROUPS_PER_STEP):
                accs[q] = accs[q] + plsc.load_gather(rows_m, [ids[q], dcol]) * wm
        for q in range(GROUPS_PER_STEP):
            out_v[pl.ds(base + q * L, L)] = accs[q]
            diff = accs[q] - rat_v[pl.ds(base + q * L, L)]
            lacc = lacc + diff * diff
        return lacc

    lacc = lax.fori_loop(0, N_STEPS, step, zero, unroll=False)
    part_v[...] = lacc

    pltpu.sync_copy(out_v, out_hbm.at[pl.ds(wid * BPW, BPW)])
    pltpu.sync_copy(part_v, part_hbm.at[wid])


@jax.jit
def _run(users_r, movies_r, ratings_r, user_table, movie_table, params):
    mesh = plsc.VectorSubcoreMesh(core_axis_name="c", subcore_axis_name="s",
                                  num_cores=NC, num_subcores=NS)
    out, part = pl.kernel(
        _sc_body,
        out_type=[
            jax.ShapeDtypeStruct((B,), jnp.float32),
            jax.ShapeDtypeStruct((NW, L), jnp.float32),
        ],
        mesh=mesh,
        compiler_params=pltpu.CompilerParams(
            needs_layout_passes=False, use_tc_tiling_on_sc=False),
        scratch_types=[
            pltpu.VMEM((N_IDX, IDX_CHUNK), jnp.int32),
            pltpu.VMEM((N_IDX, IDX_CHUNK), jnp.int32),
            pltpu.VMEM((BPW, D), jnp.float32),
            pltpu.VMEM((BPW, D), jnp.float32),
            pltpu.VMEM((BPW,), jnp.float32),
            pltpu.VMEM((BPW,), jnp.float32),
            pltpu.VMEM(((2 * D + 8) * L,), jnp.float32),
            pltpu.VMEM((L,), jnp.float32),
            pltpu.SemaphoreType.DMA,
        ],
    )(users_r, movies_r, ratings_r, user_table, movie_table, params)
    output = out.reshape(B, 1)
    loss = jnp.sum(part) * (1.0 / B)
    return output, loss


def kernel(users, movies, ratings, user_table, movie_table, W, b):
    users_r = users.reshape(NW, N_IDX, IDX_CHUNK)
    movies_r = movies.reshape(NW, N_IDX, IDX_CHUNK)
    ratings_r = ratings.reshape(NW, BPW)
    params = jnp.concatenate(
        [W.reshape(2 * D), b.reshape(1), jnp.zeros((7,), jnp.float32)])
    params_bc = jnp.broadcast_to(params[:, None], (2 * D + 8, L)).reshape(-1)
    return _run(users_r, movies_r, ratings_r, user_table, movie_table, params_bc)


# speedup vs baseline: 1.0104x; 1.0104x over previous
import functools

import jax
import jax.numpy as jnp
from jax import lax
from jax.experimental import pallas as pl
from jax.experimental.pallas import tpu as pltpu
from jax.experimental.pallas import tpu_sc as plsc

NC = 2
NS = 16
L = 16
NW = NC * NS

B = 16384
D = 64
BPW = B // NW
HALF = 256
GROUPS_PER_STEP = 4
ROWS_PER_STEP = GROUPS_PER_STEP * L
N_STEPS = HALF // ROWS_PER_STEP
PV = 3072


def _sc_body(idx2u_hbm, idx2m_hbm, pari_hbm, rat_hbm, utab_hbm, mtab_hbm,
             params_hbm, out_hbm,
             idx2u_v, idx2m_v, rowsA, rowsB, paru_v, parm_v, rat_v,
             out1d, out_pl, params_v, semA, semB):
    wid = lax.axis_index("s") * NC + lax.axis_index("c")

    pltpu.sync_copy(params_hbm, params_v)
    pltpu.sync_copy(idx2u_hbm.at[wid], idx2u_v)
    pltpu.sync_copy(idx2m_hbm.at[wid], idx2m_v)
    for j in range(4):
        pltpu.sync_copy(pari_hbm.at[wid].at[j], paru_v.at[pl.ds(j * 128, 128)])
        pltpu.sync_copy(pari_hbm.at[wid].at[4 + j],
                        parm_v.at[pl.ds(j * 128, 128)])
        pltpu.sync_copy(rat_hbm.at[wid].at[j], rat_v.at[pl.ds(j * 128, 128)])

    def fire(tab, idx_v, j0, rows, sem):
        c0 = pltpu.async_copy(tab.at[idx_v.at[j0]],
                              rows.at[pl.ds(0, 128)], sem)
        c1 = pltpu.async_copy(tab.at[idx_v.at[j0 + 1]],
                              rows.at[pl.ds(128, 128)], sem)
        return c0, c1

    iota = lax.iota(jnp.int32, L)
    zero = jnp.zeros((L,), jnp.float32)
    bias = params_v[pl.ds(2 * D * L, L)]

    def wvec(d):
        return params_v[pl.ds(d * L, L)]

    def batch(rows, par_v, phase, poff, first, lacc_in):
        def step(c, lacc):
            base = c * ROWS_PER_STEP
            ids = [base + q * L + iota for q in range(GROUPS_PER_STEP)]
            pars = [par_v[pl.ds(phase * HALF + base + q * L, L)] * 64
                    for q in range(GROUPS_PER_STEP)]
            if first:
                accs = [bias for _ in range(GROUPS_PER_STEP)]
            else:
                accs = [out1d[pl.ds(phase * HALF + base + q * L, L)]
                        for q in range(GROUPS_PER_STEP)]
            for d in range(D):
                w = wvec(poff + d)
                for q in range(GROUPS_PER_STEP):
                    dcol = pars[q] + d
                    accs[q] = accs[q] + plsc.load_gather(
                        rows, [ids[q], dcol]) * w
            for q in range(GROUPS_PER_STEP):
                off = phase * HALF + base + q * L
                out1d[pl.ds(off, L)] = accs[q]
                if not first:
                    diff = accs[q] - rat_v[pl.ds(off, L)]
                    lacc = lacc + diff * diff
            return lacc

        return lax.fori_loop(0, N_STEPS, step, lacc_in, unroll=False)

    u0 = fire(utab_hbm, idx2u_v, 0, rowsA, semA)
    u1 = fire(utab_hbm, idx2u_v, 2, rowsB, semB)
    u0[0].wait(); u0[1].wait()
    batch(rowsA, paru_v, 0, 0, True, zero)
    m0 = fire(mtab_hbm, idx2m_v, 0, rowsA, semA)
    u1[0].wait(); u1[1].wait()
    batch(rowsB, paru_v, 1, 0, True, zero)
    m1 = fire(mtab_hbm, idx2m_v, 2, rowsB, semB)
    m0[0].wait(); m0[1].wait()
    lacc = batch(rowsA, parm_v, 0, D, False, zero)
    m1[0].wait(); m1[1].wait()
    lacc = batch(rowsB, parm_v, 1, D, False, lacc)

    for j in range(4):
        pltpu.sync_copy(out1d.at[pl.ds(j * 128, 128)], out_hbm.at[wid].at[j])
    z16 = jnp.zeros((L,), jnp.float32)
    for k in range(8):
        out_pl[pl.ds(k * L, L)] = lacc if k == 0 else z16
    pltpu.sync_copy(out_pl, out_hbm.at[wid].at[4])


@jax.jit
def _run(idx2u, idx2m, pari, rat3, utab2, mtab2, params_bc):
    mesh = plsc.VectorSubcoreMesh(core_axis_name="c", subcore_axis_name="s",
                                  num_cores=NC, num_subcores=NS)
    out3, = pl.kernel(
        _sc_body,
        out_type=[jax.ShapeDtypeStruct((NW, 8, 128), jnp.float32)],
        mesh=mesh,
        compiler_params=pltpu.CompilerParams(
            needs_layout_passes=False, use_tc_tiling_on_sc=True),
        scratch_types=[
            pltpu.VMEM((8, 128), jnp.int32),
            pltpu.VMEM((8, 128), jnp.int32),
            pltpu.VMEM((HALF, 128), jnp.float32),
            pltpu.VMEM((HALF, 128), jnp.float32),
            pltpu.VMEM((BPW,), jnp.int32),
            pltpu.VMEM((BPW,), jnp.int32),
            pltpu.VMEM((BPW,), jnp.float32),
            pltpu.VMEM((BPW,), jnp.float32),
            pltpu.VMEM((128,), jnp.float32),
            pltpu.VMEM((PV,), jnp.float32),
            pltpu.SemaphoreType.DMA,
            pltpu.SemaphoreType.DMA,
        ],
    )(idx2u, idx2m, pari, rat3, utab2, mtab2, params_bc)
    output = out3[:, :4, :].reshape(B, 1)
    loss = jnp.sum(out3[:, 4, :]) * (1.0 / B)
    return output, loss


def kernel(users, movies, ratings, user_table, movie_table, W, b):
    idx2u = jnp.pad((users >> 1).reshape(NW, 4, 128), ((0, 0), (0, 4), (0, 0)))
    idx2m = jnp.pad((movies >> 1).reshape(NW, 4, 128), ((0, 0), (0, 4), (0, 0)))
    pari = jnp.concatenate(
        [(users & 1).reshape(NW, 4, 128), (movies & 1).reshape(NW, 4, 128)],
        axis=1)
    rat3 = jnp.pad(ratings.reshape(NW, 4, 128), ((0, 0), (0, 4), (0, 0)))
    utab2 = user_table.reshape(-1, 128)
    mtab2 = movie_table.reshape(-1, 128)
    params = jnp.concatenate(
        [W.reshape(2 * D), b.reshape(1), jnp.zeros((7,), jnp.float32)])
    params_bc = jnp.pad(
        jnp.broadcast_to(params[:, None], (2 * D + 8, L)).reshape(-1),
        (0, PV - (2 * D + 8) * L))
    return _run(idx2u, idx2m, pari, rat3, utab2, mtab2, params_bc)


# speedup vs baseline: 1.0158x; 1.0053x over previous
import functools

import jax
import jax.numpy as jnp
from jax import lax
from jax.experimental import pallas as pl
from jax.experimental.pallas import tpu as pltpu
from jax.experimental.pallas import tpu_sc as plsc
from jax.experimental.layout import Format, Layout

NC = 2
NS = 16
L = 16
NW = NC * NS

B = 16384
D = 64
BPW = B // NW
HALF = 256
GROUPS_PER_STEP = 4
ROWS_PER_STEP = GROUPS_PER_STEP * L
N_STEPS = HALF // ROWS_PER_STEP
PV = 3072


def _sc_body(idx2u_hbm, idx2m_hbm, pari_hbm, rat_hbm, utab_hbm, mtab_hbm,
             params_hbm, out_hbm,
             idx2u_v, idx2m_v, rowsA, rowsB, paru_v, parm_v, rat_v,
             out1d, out_pl, params_v, semA, semB):
    wid = lax.axis_index("s") * NC + lax.axis_index("c")

    pltpu.sync_copy(params_hbm, params_v)
    pltpu.sync_copy(idx2u_hbm.at[wid], idx2u_v)
    pltpu.sync_copy(idx2m_hbm.at[wid], idx2m_v)
    for j in range(4):
        pltpu.sync_copy(pari_hbm.at[wid].at[j], paru_v.at[pl.ds(j * 128, 128)])
        pltpu.sync_copy(pari_hbm.at[wid].at[4 + j],
                        parm_v.at[pl.ds(j * 128, 128)])
        pltpu.sync_copy(rat_hbm.at[wid].at[j], rat_v.at[pl.ds(j * 128, 128)])

    def fire(tab, idx_v, j0, rows, sem):
        c0 = pltpu.async_copy(tab.at[idx_v.at[j0]],
                              rows.at[pl.ds(0, 128)], sem)
        c1 = pltpu.async_copy(tab.at[idx_v.at[j0 + 1]],
                              rows.at[pl.ds(128, 128)], sem)
        return c0, c1

    iota = lax.iota(jnp.int32, L)
    zero = jnp.zeros((L,), jnp.float32)
    bias = params_v[pl.ds(2 * D * L, L)]

    def wvec(d):
        return params_v[pl.ds(d * L, L)]

    def batch(rows, par_v, phase, poff, first, lacc_in):
        def step(c, lacc):
            base = c * ROWS_PER_STEP
            ids = [base + q * L + iota for q in range(GROUPS_PER_STEP)]
            pars = [par_v[pl.ds(phase * HALF + base + q * L, L)] * 64
                    for q in range(GROUPS_PER_STEP)]
            if first:
                accs = [bias for _ in range(GROUPS_PER_STEP)]
            else:
                accs = [out1d[pl.ds(phase * HALF + base + q * L, L)]
                        for q in range(GROUPS_PER_STEP)]
            for d in range(D):
                w = wvec(poff + d)
                for q in range(GROUPS_PER_STEP):
                    dcol = pars[q] + d
                    accs[q] = accs[q] + plsc.load_gather(
                        rows, [ids[q], dcol]) * w
            for q in range(GROUPS_PER_STEP):
                off = phase * HALF + base + q * L
                out1d[pl.ds(off, L)] = accs[q]
                if not first:
                    diff = accs[q] - rat_v[pl.ds(off, L)]
                    lacc = lacc + diff * diff
            return lacc

        return lax.fori_loop(0, N_STEPS, step, lacc_in, unroll=False)

    u0 = fire(utab_hbm, idx2u_v, 0, rowsA, semA)
    u1 = fire(utab_hbm, idx2u_v, 2, rowsB, semB)
    u0[0].wait(); u0[1].wait()
    batch(rowsA, paru_v, 0, 0, True, zero)
    m0 = fire(mtab_hbm, idx2m_v, 0, rowsA, semA)
    u1[0].wait(); u1[1].wait()
    batch(rowsB, paru_v, 1, 0, True, zero)
    m1 = fire(mtab_hbm, idx2m_v, 2, rowsB, semB)
    m0[0].wait(); m0[1].wait()
    lacc = batch(rowsA, parm_v, 0, D, False, zero)
    m1[0].wait(); m1[1].wait()
    lacc = batch(rowsB, parm_v, 1, D, False, lacc)

    for j in range(4):
        pltpu.sync_copy(out1d.at[pl.ds(j * 128, 128)], out_hbm.at[wid].at[j])
    z16 = jnp.zeros((L,), jnp.float32)
    for k in range(8):
        out_pl[pl.ds(k * L, L)] = lacc if k == 0 else z16
    pltpu.sync_copy(out_pl, out_hbm.at[wid].at[4])


@jax.jit
def _run(idx2u, idx2m, pari, rat3, utab2, mtab2, params_bc):
    mesh = plsc.VectorSubcoreMesh(core_axis_name="c", subcore_axis_name="s",
                                  num_cores=NC, num_subcores=NS)
    out3, = pl.kernel(
        _sc_body,
        out_type=[jax.ShapeDtypeStruct((NW, 8, 128), jnp.float32)],
        mesh=mesh,
        compiler_params=pltpu.CompilerParams(
            needs_layout_passes=False, use_tc_tiling_on_sc=True),
        scratch_types=[
            pltpu.VMEM((8, 128), jnp.int32),
            pltpu.VMEM((8, 128), jnp.int32),
            pltpu.VMEM((HALF, 128), jnp.float32),
            pltpu.VMEM((HALF, 128), jnp.float32),
            pltpu.VMEM((BPW,), jnp.int32),
            pltpu.VMEM((BPW,), jnp.int32),
            pltpu.VMEM((BPW,), jnp.float32),
            pltpu.VMEM((BPW,), jnp.float32),
            pltpu.VMEM((128,), jnp.float32),
            pltpu.VMEM((PV,), jnp.float32),
            pltpu.SemaphoreType.DMA,
            pltpu.SemaphoreType.DMA,
        ],
    )(idx2u, idx2m, pari, rat3, utab2, mtab2, params_bc)
    output = out3[:, :4, :].reshape(B, 1)
    loss = jnp.sum(out3[:, 4, :]) * (1.0 / B)
    return output, loss


def kernel(users, movies, ratings, user_table, movie_table, W, b):
    idx2u = jnp.pad((users >> 1).reshape(NW, 4, 128), ((0, 0), (0, 4), (0, 0)))
    idx2m = jnp.pad((movies >> 1).reshape(NW, 4, 128), ((0, 0), (0, 4), (0, 0)))
    pari = jnp.concatenate(
        [(users & 1).reshape(NW, 4, 128), (movies & 1).reshape(NW, 4, 128)],
        axis=1)
    rat3 = jnp.pad(ratings.reshape(NW, 4, 128), ((0, 0), (0, 4), (0, 0)))
    fmt = Format(Layout(major_to_minor=(0, 1), tiling=((8, 128),)),
          jax.sharding.SingleDeviceSharding(jax.devices()[0]))
    utab2 = jax.device_put(user_table, fmt).reshape(-1, 128)
    mtab2 = jax.device_put(movie_table, fmt).reshape(-1, 128)
    params = jnp.concatenate(
        [W.reshape(2 * D), b.reshape(1), jnp.zeros((7,), jnp.float32)])
    params_bc = jnp.pad(
        jnp.broadcast_to(params[:, None], (2 * D + 8, L)).reshape(-1),
        (0, PV - (2 * D + 8) * L))
    return _run(idx2u, idx2m, pari, rat3, utab2, mtab2, params_bc)


# speedup vs baseline: 1.1149x; 1.0976x over previous
import functools

import jax
import jax.numpy as jnp
from jax import lax
from jax.experimental import pallas as pl
from jax.experimental.pallas import tpu as pltpu
from jax.experimental.pallas import tpu_sc as plsc
from jax.experimental.layout import Format, Layout

NC = 2
NS = 16
L = 16
NW = NC * NS

B = 16384
D = 64
BPW = B // NW
HALF = 256
GROUPS_PER_STEP = 4
ROWS_PER_STEP = GROUPS_PER_STEP * L
N_STEPS = HALF // ROWS_PER_STEP
PV = 3072


def _sc_body(idx2u_hbm, idx2m_hbm, rat_hbm, utab_hbm, mtab_hbm,
             params_hbm, out_hbm,
             idx2u_v, idx2m_v, rowsA, rowsB, rat_v,
             out1d, out_pl, params_v, semA, semB):
    wid = lax.axis_index("s") * NC + lax.axis_index("c")

    pltpu.sync_copy(params_hbm, params_v)
    pltpu.sync_copy(idx2u_hbm.at[wid], idx2u_v)
    pltpu.sync_copy(idx2m_hbm.at[wid], idx2m_v)
    for j in range(4):
        pltpu.sync_copy(rat_hbm.at[wid].at[j], rat_v.at[pl.ds(j * 128, 128)])

    def fire(tab, idx_v, j0, rows, sem):
        c0 = pltpu.async_copy(tab.at[idx_v.at[j0]],
                              rows.at[pl.ds(0, 128)], sem)
        c1 = pltpu.async_copy(tab.at[idx_v.at[j0 + 1]],
                              rows.at[pl.ds(128, 128)], sem)
        return c0, c1

    iota = lax.iota(jnp.int32, L)
    zero = jnp.zeros((L,), jnp.float32)
    bias = params_v[pl.ds(2 * D * L, L)]

    def wvec(d):
        return params_v[pl.ds(d * L, L)]

    def batch(rows, phase, poff, first, lacc_in):
        def step(c, lacc):
            base = c * ROWS_PER_STEP
            ids = [base + q * L + iota for q in range(GROUPS_PER_STEP)]
            if first:
                accs = [bias for _ in range(GROUPS_PER_STEP)]
            else:
                accs = [out1d[pl.ds(phase * HALF + base + q * L, L)]
                        for q in range(GROUPS_PER_STEP)]
            for d in range(D):
                w = wvec(poff + d)
                dcol = jnp.full((L,), d, jnp.int32)
                for q in range(GROUPS_PER_STEP):
                    accs[q] = accs[q] + plsc.load_gather(
                        rows, [ids[q], dcol]) * w
            for q in range(GROUPS_PER_STEP):
                off = phase * HALF + base + q * L
                out1d[pl.ds(off, L)] = accs[q]
                if not first:
                    diff = accs[q] - rat_v[pl.ds(off, L)]
                    lacc = lacc + diff * diff
            return lacc

        return lax.fori_loop(0, N_STEPS, step, lacc_in, unroll=False)

    u0 = fire(utab_hbm, idx2u_v, 0, rowsA, semA)
    u1 = fire(utab_hbm, idx2u_v, 2, rowsB, semB)
    u0[0].wait(); u0[1].wait()
    batch(rowsA, 0, 0, True, zero)
    m0 = fire(mtab_hbm, idx2m_v, 0, rowsA, semA)
    u1[0].wait(); u1[1].wait()
    batch(rowsB, 1, 0, True, zero)
    m1 = fire(mtab_hbm, idx2m_v, 2, rowsB, semB)
    m0[0].wait(); m0[1].wait()
    lacc = batch(rowsA, 0, D, False, zero)
    m1[0].wait(); m1[1].wait()
    lacc = batch(rowsB, 1, D, False, lacc)

    for j in range(4):
        pltpu.sync_copy(out1d.at[pl.ds(j * 128, 128)], out_hbm.at[wid].at[j])
    z16 = jnp.zeros((L,), jnp.float32)
    for k in range(8):
        out_pl[pl.ds(k * L, L)] = lacc if k == 0 else z16
    pltpu.sync_copy(out_pl, out_hbm.at[wid].at[4])


@jax.jit
def _run(idx2u, idx2m, rat3, utab2, mtab2, params_bc):
    mesh = plsc.VectorSubcoreMesh(core_axis_name="c", subcore_axis_name="s",
                                  num_cores=NC, num_subcores=NS)
    out3, = pl.kernel(
        _sc_body,
        out_type=[jax.ShapeDtypeStruct((NW, 8, 128), jnp.float32)],
        mesh=mesh,
        compiler_params=pltpu.CompilerParams(
            needs_layout_passes=False, use_tc_tiling_on_sc=True),
        scratch_types=[
            pltpu.VMEM((8, 128), jnp.int32),
            pltpu.VMEM((8, 128), jnp.int32),
            pltpu.VMEM((HALF, 128), jnp.float32),
            pltpu.VMEM((HALF, 128), jnp.float32),
            pltpu.VMEM((BPW,), jnp.float32),
            pltpu.VMEM((BPW,), jnp.float32),
            pltpu.VMEM((128,), jnp.float32),
            pltpu.VMEM((PV,), jnp.float32),
            pltpu.SemaphoreType.DMA,
            pltpu.SemaphoreType.DMA,
        ],
    )(idx2u, idx2m, rat3, utab2, mtab2, params_bc)
    output = out3[:, :4, :].reshape(B, 1)
    loss = jnp.sum(out3[:, 4, :]) * (1.0 / B)
    return output, loss


def kernel(users, movies, ratings, user_table, movie_table, W, b):
    idx2u = jnp.pad(users.reshape(NW, 4, 128), ((0, 0), (0, 4), (0, 0)))
    idx2m = jnp.pad(movies.reshape(NW, 4, 128), ((0, 0), (0, 4), (0, 0)))
    rat3 = jnp.pad(ratings.reshape(NW, 4, 128), ((0, 0), (0, 4), (0, 0)))
    utab2 = jnp.pad(user_table, ((0, 0), (0, 128 - D)))
    mtab2 = jnp.pad(movie_table, ((0, 0), (0, 128 - D)))
    params = jnp.concatenate(
        [W.reshape(2 * D), b.reshape(1), jnp.zeros((7,), jnp.float32)])
    params_bc = jnp.pad(
        jnp.broadcast_to(params[:, None], (2 * D + 8, L)).reshape(-1),
        (0, PV - (2 * D + 8) * L))
    return _run(idx2u, idx2m, rat3, utab2, mtab2, params_bc)


# speedup vs baseline: 1.1249x; 1.0090x over previous
import functools

import jax
import jax.numpy as jnp
from jax import lax
from jax.experimental import pallas as pl
from jax.experimental.pallas import tpu as pltpu
from jax.experimental.pallas import tpu_sc as plsc
from jax.experimental.layout import Format, Layout

NC = 2
NS = 16
L = 16
NW = NC * NS

B = 16384
D = 64
BPW = B // NW
HALF = 256
GROUPS_PER_STEP = 4
ROWS_PER_STEP = GROUPS_PER_STEP * L
N_STEPS = HALF // ROWS_PER_STEP
PV = 3072


def _sc_body(idx2u_hbm, idx2m_hbm, rat_hbm, ctab_hbm,
             params_hbm, out_hbm,
             idx2u_v, idx2m_v, rowsA, rowsB, rat_v,
             out1d, out_pl, params_v, semA, semB):
    wid = lax.axis_index("s") * NC + lax.axis_index("c")

    pltpu.sync_copy(params_hbm, params_v)
    pltpu.sync_copy(idx2u_hbm.at[wid], idx2u_v)
    pltpu.sync_copy(idx2m_hbm.at[wid], idx2m_v)
    for j in range(4):
        pltpu.sync_copy(rat_hbm.at[wid].at[j], rat_v.at[pl.ds(j * 128, 128)])

    def fire(tab, idx_v, j0, rows, sem):
        c0 = pltpu.async_copy(tab.at[idx_v.at[j0]],
                              rows.at[pl.ds(0, 128)], sem)
        c1 = pltpu.async_copy(tab.at[idx_v.at[j0 + 1]],
                              rows.at[pl.ds(128, 128)], sem)
        return c0, c1

    iota = lax.iota(jnp.int32, L)
    zero = jnp.zeros((L,), jnp.float32)
    bias = params_v[pl.ds(2 * D * L, L)]

    def wvec(d):
        return params_v[pl.ds(d * L, L)]

    def batch(rows, phase, poff, coff, first, lacc_in):
        def step(c, lacc):
            base = c * ROWS_PER_STEP
            ids = [base + q * L + iota for q in range(GROUPS_PER_STEP)]
            if first:
                accs = [bias for _ in range(GROUPS_PER_STEP)]
            else:
                accs = [out1d[pl.ds(phase * HALF + base + q * L, L)]
                        for q in range(GROUPS_PER_STEP)]
            for d in range(D):
                w = wvec(poff + d)
                dcol = jnp.full((L,), coff + d, jnp.int32)
                for q in range(GROUPS_PER_STEP):
                    accs[q] = accs[q] + plsc.load_gather(
                        rows, [ids[q], dcol]) * w
            for q in range(GROUPS_PER_STEP):
                off = phase * HALF + base + q * L
                out1d[pl.ds(off, L)] = accs[q]
                if not first:
                    diff = accs[q] - rat_v[pl.ds(off, L)]
                    lacc = lacc + diff * diff
            return lacc

        return lax.fori_loop(0, N_STEPS, step, lacc_in, unroll=False)

    u0 = fire(ctab_hbm, idx2u_v, 0, rowsA, semA)
    u1 = fire(ctab_hbm, idx2u_v, 2, rowsB, semB)
    u0[0].wait(); u0[1].wait()
    batch(rowsA, 0, 0, 0, True, zero)
    m0 = fire(ctab_hbm, idx2m_v, 0, rowsA, semA)
    u1[0].wait(); u1[1].wait()
    batch(rowsB, 1, 0, 0, True, zero)
    m1 = fire(ctab_hbm, idx2m_v, 2, rowsB, semB)
    m0[0].wait(); m0[1].wait()
    lacc = batch(rowsA, 0, D, D, False, zero)
    m1[0].wait(); m1[1].wait()
    lacc = batch(rowsB, 1, D, D, False, lacc)

    for j in range(4):
        pltpu.sync_copy(out1d.at[pl.ds(j * 128, 128)], out_hbm.at[wid].at[j])
    z16 = jnp.zeros((L,), jnp.float32)
    for k in range(8):
        out_pl[pl.ds(k * L, L)] = lacc if k == 0 else z16
    pltpu.sync_copy(out_pl, out_hbm.at[wid].at[4])


@jax.jit
def _run(idx2u, idx2m, rat3, ctab, params_bc):
    mesh = plsc.VectorSubcoreMesh(core_axis_name="c", subcore_axis_name="s",
                                  num_cores=NC, num_subcores=NS)
    out3, = pl.kernel(
        _sc_body,
        out_type=[jax.ShapeDtypeStruct((NW, 8, 128), jnp.float32)],
        mesh=mesh,
        compiler_params=pltpu.CompilerParams(
            needs_layout_passes=False, use_tc_tiling_on_sc=True),
        scratch_types=[
            pltpu.VMEM((8, 128), jnp.int32),
            pltpu.VMEM((8, 128), jnp.int32),
            pltpu.VMEM((HALF, 128), jnp.float32),
            pltpu.VMEM((HALF, 128), jnp.float32),
            pltpu.VMEM((BPW,), jnp.float32),
            pltpu.VMEM((BPW,), jnp.float32),
            pltpu.VMEM((128,), jnp.float32),
            pltpu.VMEM((PV,), jnp.float32),
            pltpu.SemaphoreType.DMA,
            pltpu.SemaphoreType.DMA,
        ],
    )(idx2u, idx2m, rat3, ctab, params_bc)
    output = out3[:, :4, :].reshape(B, 1)
    loss = jnp.sum(out3[:, 4, :]) * (1.0 / B)
    return output, loss


def kernel(users, movies, ratings, user_table, movie_table, W, b):
    idx2u = jnp.pad(users.reshape(NW, 4, 128), ((0, 0), (0, 4), (0, 0)))
    idx2m = jnp.pad(movies.reshape(NW, 4, 128), ((0, 0), (0, 4), (0, 0)))
    rat3 = jnp.pad(ratings.reshape(NW, 4, 128), ((0, 0), (0, 4), (0, 0)))
    ctab = jnp.concatenate(
        [user_table,
         jnp.pad(movie_table, ((0, user_table.shape[0] - movie_table.shape[0]),
                               (0, 0)))], axis=1)
    params = jnp.concatenate(
        [W.reshape(2 * D), b.reshape(1), jnp.zeros((7,), jnp.float32)])
    params_bc = jnp.pad(
        jnp.broadcast_to(params[:, None], (2 * D + 8, L)).reshape(-1),
        (0, PV - (2 * D + 8) * L))
    return _run(idx2u, idx2m, rat3, ctab, params_bc)
